# baseline (speedup 1.0000x reference)
import functools

import jax
import jax.numpy as jnp
from jax import lax
from jax.experimental import pallas as pl
from jax.experimental.pallas import tpu as pltpu
from jax.experimental.pallas import tpu_sc as plsc

_B, _S = 4, 4096
_D = 128
_HALF = _D // 2
_N = _B * _S
_CHUNK = 128
_FREQ = 10000.0
_BLK = 512

_NBUF = 4
_LOOKAHEAD = 3


@functools.cache
def _build_sc_gather():
    mesh = plsc.VectorSubcoreMesh(core_axis_name="c", subcore_axis_name="s")

    @functools.partial(
        pl.kernel,
        out_type=jax.ShapeDtypeStruct((_N, _D), jnp.float32),
        mesh=mesh,
        scratch_types=[
            pltpu.VMEM((_B, _CHUNK), jnp.int32),
            pltpu.VMEM((_NBUF, _CHUNK, _D), jnp.float32),
            pltpu.SemaphoreType.DMA((_NBUF,)),
            pltpu.SemaphoreType.DMA((_NBUF,)),
        ],
    )
    def gather_kernel(tab_hbm, idx_hbm, out, idx_v, bufs, gsem, osem):
        wid = lax.axis_index("s") * mesh.num_cores + lax.axis_index("c")
        col = wid * _CHUNK
        pltpu.sync_copy(
            idx_hbm.at[pl.ds(0, _B), pl.ds(col, _CHUNK)], idx_v)

        g, o = {}, {}

        def fire_out(t):
            g[t].wait()
            o[t] = pltpu.async_copy(
                bufs.at[t],
                out.at[pl.ds(t * _S + col, _CHUNK)],
                osem.at[t])

        for s in range(_B):
            g[s] = pltpu.async_copy(
                tab_hbm.at[idx_v.at[s]], bufs.at[s], gsem.at[s])
            if s >= _LOOKAHEAD:
                fire_out(s - _LOOKAHEAD)
        for t in range(_B - _LOOKAHEAD, _B):
            fire_out(t)
        for t in range(_B):
            o[t].wait()

    return gather_kernel


def _trig_block(p_ref, invf_ref, sin_ref):
    p = p_ref[...].reshape(_B * _BLK).astype(jnp.float32)
    invf = invf_ref[...]
    half = jnp.sin(p[:, None] * invf)
    full = jnp.concatenate([half, half], axis=1)
    sin_ref[...] = full.reshape(_B, _BLK, _D)


@functools.cache
def _build_tc_trig():
    return pl.pallas_call(
        _trig_block,
        grid=(_S // _BLK,),
        in_specs=[
            pl.BlockSpec((_B, _BLK), lambda s: (0, s)),
            pl.BlockSpec((1, _HALF), lambda s: (0, 0)),
        ],
        out_specs=pl.BlockSpec((_B, _BLK, _D), lambda s: (0, s, 0)),
        out_shape=jax.ShapeDtypeStruct((_B, _S, _D), jnp.float32),
    )


def kernel(x, position_ids, cos_cached, sin_cached):
    inv_freq = 1.0 / (_FREQ ** (jnp.arange(0, _D, 2, dtype=jnp.float32) / _D))
    invf = inv_freq.reshape(1, _HALF)
    cos_out = _build_sc_gather()(cos_cached, position_ids)
    sin_out = _build_tc_trig()(position_ids, invf)
    return (cos_out.reshape(_B, _S, _D), sin_out)


# speedup vs baseline: 1.5153x; 1.5153x over previous
import functools

import jax
import jax.numpy as jnp
from jax import lax
from jax.experimental import pallas as pl
from jax.experimental.pallas import tpu as pltpu
from jax.experimental.pallas import tpu_sc as plsc

_B, _S = 4, 4096
_D = 128
_HALF = _D // 2
_N = _B * _S
_CHUNK = 128
_FREQ = 10000.0
_BLK = 512

_NBUF = 4
_LOOKAHEAD = 3


@functools.cache
def _build_sc_gather():
    mesh = plsc.VectorSubcoreMesh(core_axis_name="c", subcore_axis_name="s")

    @functools.partial(
        pl.kernel,
        out_type=jax.ShapeDtypeStruct((_N, _D), jnp.float32),
        mesh=mesh,
        scratch_types=[
            pltpu.VMEM((_B, _CHUNK), jnp.int32),
            pltpu.VMEM((_NBUF, _CHUNK, _D), jnp.float32),
            pltpu.SemaphoreType.DMA((_NBUF,)),
            pltpu.SemaphoreType.DMA((_NBUF,)),
        ],
    )
    def gather_kernel(tab_hbm, idx_hbm, out, idx_v, bufs, gsem, osem):
        wid = lax.axis_index("s") * mesh.num_cores + lax.axis_index("c")
        col = wid * _CHUNK
        pltpu.sync_copy(
            idx_hbm.at[pl.ds(0, _B), pl.ds(col, _CHUNK)], idx_v)

        g, o = {}, {}

        def fire_out(t):
            g[t].wait()
            o[t] = pltpu.async_copy(
                bufs.at[t],
                out.at[pl.ds(t * _S + col, _CHUNK)],
                osem.at[t])

        for s in range(_B):
            g[s] = pltpu.async_copy(
                tab_hbm.at[idx_v.at[s]], bufs.at[s], gsem.at[s])
            if s >= _LOOKAHEAD:
                fire_out(s - _LOOKAHEAD)
        for t in range(_B - _LOOKAHEAD, _B):
            fire_out(t)
        for t in range(_B):
            o[t].wait()

    return gather_kernel


def _trig_block(p_ref, ta_ref, tb_ref, sin_ref):
    p = p_ref[...].reshape(_B * _BLK)
    a = p >> 6
    b = p & 63
    ids = lax.broadcasted_iota(jnp.int32, (_B * _BLK, _D), 1)
    oh_a = (a[:, None] == ids).astype(jnp.float32)
    oh_b = (b[:, None] == ids).astype(jnp.float32)
    ra = jnp.dot(oh_a, ta_ref[...], preferred_element_type=jnp.float32)
    rb = jnp.dot(oh_b, tb_ref[...], preferred_element_type=jnp.float32)
    sin_a, cos_a = ra[:, :_HALF], ra[:, _HALF:]
    sin_b, cos_b = rb[:, :_HALF], rb[:, _HALF:]
    half = sin_a * cos_b + cos_a * sin_b
    full = jnp.concatenate([half, half], axis=1)
    sin_ref[...] = full.reshape(_B, _BLK, _D)


@functools.cache
def _build_tc_trig():
    return pl.pallas_call(
        _trig_block,
        grid=(_S // _BLK,),
        in_specs=[
            pl.BlockSpec((_B, _BLK), lambda s: (0, s)),
            pl.BlockSpec((_D, _D), lambda s: (0, 0)),
            pl.BlockSpec((_D, _D), lambda s: (0, 0)),
        ],
        out_specs=pl.BlockSpec((_B, _BLK, _D), lambda s: (0, s, 0)),
        out_shape=jax.ShapeDtypeStruct((_B, _S, _D), jnp.float32),
    )


def kernel(x, position_ids, cos_cached, sin_cached):
    inv_freq = 1.0 / (_FREQ ** (jnp.arange(0, _D, 2, dtype=jnp.float32) / _D))
    k = jnp.arange(_D, dtype=jnp.float32)
    arg_a = jnp.outer(64.0 * k, inv_freq)
    arg_b = jnp.outer(k, inv_freq)
    ta = jnp.concatenate([jnp.sin(arg_a), jnp.cos(arg_a)], axis=1)
    tb = jnp.concatenate([jnp.sin(arg_b), jnp.cos(arg_b)], axis=1)
    cos_out = _build_sc_gather()(cos_cached, position_ids)
    sin_out = _build_tc_trig()(position_ids, ta, tb)
    return (cos_out.reshape(_B, _S, _D), sin_out)


# speedup vs baseline: 1.5285x; 1.0087x over previous
import functools

import jax
import jax.numpy as jnp
from jax import lax
from jax.experimental import pallas as pl
from jax.experimental.pallas import tpu as pltpu
from jax.experimental.pallas import tpu_sc as plsc

_B, _S = 4, 4096
_D = 128
_HALF = _D // 2
_N = _B * _S
_CHUNK = 128
_FREQ = 10000.0
_BLK = 1024

_NBUF = 4
_LOOKAHEAD = 3


@functools.cache
def _build_sc_gather():
    mesh = plsc.VectorSubcoreMesh(core_axis_name="c", subcore_axis_name="s")

    @functools.partial(
        pl.kernel,
        out_type=jax.ShapeDtypeStruct((_N, _D), jnp.float32),
        mesh=mesh,
        scratch_types=[
            pltpu.VMEM((_B, _CHUNK), jnp.int32),
            pltpu.VMEM((_NBUF, _CHUNK, _D), jnp.float32),
            pltpu.SemaphoreType.DMA((_NBUF,)),
            pltpu.SemaphoreType.DMA((_NBUF,)),
        ],
    )
    def gather_kernel(tab_hbm, idx_hbm, out, idx_v, bufs, gsem, osem):
        wid = lax.axis_index("s") * mesh.num_cores + lax.axis_index("c")
        col = wid * _CHUNK
        pltpu.sync_copy(
            idx_hbm.at[pl.ds(0, _B), pl.ds(col, _CHUNK)], idx_v)

        g, o = {}, {}

        def fire_out(t):
            g[t].wait()
            o[t] = pltpu.async_copy(
                bufs.at[t],
                out.at[pl.ds(t * _S + col, _CHUNK)],
                osem.at[t])

        for s in range(_B):
            g[s] = pltpu.async_copy(
                tab_hbm.at[idx_v.at[s]], bufs.at[s], gsem.at[s])
            if s >= _LOOKAHEAD:
                fire_out(s - _LOOKAHEAD)
        for t in range(_B - _LOOKAHEAD, _B):
            fire_out(t)
        for t in range(_B):
            o[t].wait()

    return gather_kernel


def _trig_block(p_ref, ta_ref, tb_ref, sin_ref):
    p = p_ref[...].reshape(_B * _BLK)
    a = p >> 6
    b = p & 63
    ids = lax.broadcasted_iota(jnp.int32, (_B * _BLK, _D), 1)
    oh_a = (a[:, None] == ids).astype(jnp.float32)
    oh_b = (b[:, None] == ids).astype(jnp.float32)
    ra = jnp.dot(oh_a, ta_ref[...], preferred_element_type=jnp.float32)
    rb = jnp.dot(oh_b, tb_ref[...], preferred_element_type=jnp.float32)
    sin_a, cos_a = ra[:, :_HALF], ra[:, _HALF:]
    sin_b, cos_b = rb[:, :_HALF], rb[:, _HALF:]
    half = sin_a * cos_b + cos_a * sin_b
    full = jnp.concatenate([half, half], axis=1)
    sin_ref[...] = full.reshape(_B, _BLK, _D)


@functools.cache
def _build_tc_trig():
    return pl.pallas_call(
        _trig_block,
        grid=(_S // _BLK,),
        in_specs=[
            pl.BlockSpec((_B, _BLK), lambda s: (0, s)),
            pl.BlockSpec((_D, _D), lambda s: (0, 0)),
            pl.BlockSpec((_D, _D), lambda s: (0, 0)),
        ],
        out_specs=pl.BlockSpec((_B, _BLK, _D), lambda s: (0, s, 0)),
        out_shape=jax.ShapeDtypeStruct((_B, _S, _D), jnp.float32),
    )


def kernel(x, position_ids, cos_cached, sin_cached):
    inv_freq = 1.0 / (_FREQ ** (jnp.arange(0, _D, 2, dtype=jnp.float32) / _D))
    k = jnp.arange(_D, dtype=jnp.float32)
    arg_a = jnp.outer(64.0 * k, inv_freq)
    arg_b = jnp.outer(k, inv_freq)
    ta = jnp.concatenate([jnp.sin(arg_a), jnp.cos(arg_a)], axis=1)
    tb = jnp.concatenate([jnp.sin(arg_b), jnp.cos(arg_b)], axis=1)
    cos_out = _build_sc_gather()(cos_cached, position_ids)
    sin_out = _build_tc_trig()(position_ids, ta, tb)
    return (cos_out.reshape(_B, _S, _D), sin_out)
